# Initial kernel scaffold; baseline (speedup 1.0000x reference)
import jax
import jax.numpy as jnp
from jax.experimental import pallas as pl


def kernel(atom_emb, bond_tabs, W1s, b1s, g1s, bt1s, W2s, b2s, epss, g2s, bt2s, x, edge_attr, edge_index, batch):
    raise NotImplementedError("write your pallas kernel here")


# speedup vs baseline: 5.9822x; 5.9822x over previous
import functools

import jax
import jax.numpy as jnp
from jax import lax
from jax.experimental import pallas as pl
from jax.experimental.pallas import tpu as pltpu
from jax.experimental.pallas import tpu_sc as plsc

NC = 2
NS = 16
BN = 2000


def _dot_x3(a, b):
    f = jnp.float32
    ah = a.astype(jnp.bfloat16)
    al = (a - ah.astype(f)).astype(jnp.bfloat16)
    bh = b.astype(jnp.bfloat16)
    bl = (b - bh.astype(f)).astype(jnp.bfloat16)
    dn = (((1,), (0,)), ((), ()))
    return (lax.dot_general(al, bh, dn, preferred_element_type=f)
            + lax.dot_general(ah, bl, dn, preferred_element_type=f)
            + lax.dot_general(ah, bh, dn, preferred_element_type=f))


def _edge_pass(T_flat, idx2p, srcp, n, nacc, ng, gb, nblk):
    mesh = plsc.VectorSubcoreMesh(
        core_axis_name="c", subcore_axis_name="s", num_cores=NC, num_subcores=NS
    )
    nz = nacc // NS
    ntail = n - (NS - 1) * nz
    assert 0 < ntail <= nz and ntail % 8 == 0 and nz % 8 == 0

    nslot = 4

    @functools.partial(
        pl.kernel,
        out_type=jax.ShapeDtypeStruct((NC, n, 32), jnp.float32),
        mesh=mesh,
        compiler_params=pltpu.CompilerParams(use_tc_tiling_on_sc=False),
        scratch_types=[
            pltpu.VMEM((gb, 128), jnp.int32),
            pltpu.VMEM((gb, 128), jnp.int32),
            pltpu.VMEM((nslot * 128, 32), jnp.float32),
            pltpu.VMEM_SHARED((nacc, 32), jnp.float32),
            pltpu.SemaphoreType.DMA,
            pltpu.SemaphoreType.DMA,
        ],
    )
    def edge_kernel(t_hbm, idx_hbm, src_hbm, agg_hbm,
                    idx_blk, src_blk, rows, acc, gsem, ssem):
        c = lax.axis_index("c")
        s = lax.axis_index("s")

        zero16 = jnp.zeros((16,), jnp.float32)

        def zrow(i, carry):
            rows[i, pl.ds(0, 16)] = zero16
            rows[i, pl.ds(16, 16)] = zero16
            return carry

        lax.fori_loop(0, nslot * 128, zrow, 0)

        nzc = nslot * 128

        def zacc(k, carry):
            pltpu.sync_copy(rows, acc.at[pl.ds(s * nz + k * nzc, nzc)])
            return carry

        lax.fori_loop(0, nz // nzc, zacc, 0)
        if nz % nzc:
            pltpu.sync_copy(rows.at[pl.ds(0, nz % nzc)],
                            acc.at[pl.ds(s * nz + (nz // nzc) * nzc, nz % nzc)])
        plsc.subcore_barrier()

        def block(bi, carry):
            g0 = bi * gb
            pltpu.sync_copy(idx_hbm.at[c, s, pl.ds(g0, gb)], idx_blk)
            pltpu.sync_copy(src_hbm.at[s, pl.ds(g0, gb)], src_blk)
            for half in range(gb // nslot):
                gathers = [
                    pltpu.async_copy(t_hbm.at[idx_blk.at[half * nslot + j]],
                                     rows.at[pl.ds(j * 128, 128)], gsem)
                    for j in range(nslot)
                ]
                for d in gathers:
                    d.wait()
                scatters = [
                    pltpu.async_copy(rows.at[pl.ds(j * 128, 128)],
                                     acc.at[src_blk.at[half * nslot + j]],
                                     ssem, add=True)
                    for j in range(nslot)
                ]
                for d in scatters:
                    d.wait()
            return carry

        lax.fori_loop(0, nblk, block, 0)
        plsc.subcore_barrier()
        pltpu.sync_copy(acc.at[pl.ds(s * nz, ntail)],
                        agg_hbm.at[c, pl.ds(s * nz, ntail)])

        @pl.when(s < NS - 1)
        def _():
            pltpu.sync_copy(acc.at[pl.ds(s * nz + ntail, nz - ntail)],
                            agg_hbm.at[c, pl.ds(s * nz + ntail, nz - ntail)])

    return edge_kernel(T_flat, idx2p, srcp)


def _expand0(x3, atom_emb, bond_half, n):
    nb = x3.shape[0]
    av, d = atom_emb.shape

    def kern(x_ref, emb_ref, bond_ref, h_ref, t_ref):
        xb = x_ref[0, 0, :]
        onehot = (lax.broadcasted_iota(jnp.int32, (av, BN), 0)
                  == xb[None, :]).astype(jnp.float32)
        hb = lax.dot_general(onehot, emb_ref[...], (((0,), (0,)), ((), ())),
                             preferred_element_type=jnp.float32, precision=lax.Precision.HIGHEST)
        h_ref[...] = hb
        bond = bond_ref[...]
        for cb in range(10):
            half = hb[:, 32 * (cb // 5):32 * (cb // 5) + 32]
            t_ref[cb] = jnp.maximum(half + bond[cb][None, :], 0.0)

    return pl.pallas_call(
        kern,
        grid=(nb,),
        in_specs=[
            pl.BlockSpec((1, 1, BN), lambda i: (i, 0, 0)),
            pl.BlockSpec((av, d), lambda i: (0, 0)),
            pl.BlockSpec((10, 32), lambda i: (0, 0)),
        ],
        out_specs=[
            pl.BlockSpec((BN, d), lambda i: (i, 0)),
            pl.BlockSpec((10, BN, 32), lambda i: (0, i, 0)),
        ],
        out_shape=[
            jax.ShapeDtypeStruct((n, d), jnp.float32),
            jax.ShapeDtypeStruct((10, n, 32), jnp.float32),
        ],
    )(x3, atom_emb, bond_half)


def _pass_a(h, agg, w1, b1, eps, n):
    nb = n // BN

    def kern(h_ref, agg_ref, w1_ref, b1_ref, eps_ref, y1_ref, sh_ref, s_ref,
             q_ref):
        e = 1.0 + eps_ref[0, 0]
        z = e * h_ref[...] + jnp.concatenate([agg_ref[0], agg_ref[1]], axis=1)
        y1 = jnp.dot(z, w1_ref[...], preferred_element_type=jnp.float32) + b1_ref[...]
        y1_ref[...] = y1

        @pl.when(pl.program_id(0) == 0)
        def _():
            sh_ref[...] = jnp.sum(y1, axis=0, keepdims=True) / BN
            s_ref[...] = jnp.zeros_like(s_ref)
            q_ref[...] = jnp.zeros_like(q_ref)

        yc = y1 - sh_ref[...]
        s_ref[...] += jnp.sum(yc, axis=0, keepdims=True)
        q_ref[...] += jnp.sum(yc * yc, axis=0, keepdims=True)

    return pl.pallas_call(
        kern,
        grid=(nb,),
        in_specs=[
            pl.BlockSpec((BN, 64), lambda i: (i, 0)),
            pl.BlockSpec((2, BN, 32), lambda i: (0, i, 0)),
            pl.BlockSpec((64, 128), lambda i: (0, 0)),
            pl.BlockSpec((1, 128), lambda i: (0, 0)),
            pl.BlockSpec((1, 1), lambda i: (0, 0)),
        ],
        out_specs=[
            pl.BlockSpec((BN, 128), lambda i: (i, 0)),
            pl.BlockSpec((1, 128), lambda i: (0, 0)),
            pl.BlockSpec((1, 128), lambda i: (0, 0)),
            pl.BlockSpec((1, 128), lambda i: (0, 0)),
        ],
        out_shape=[
            jax.ShapeDtypeStruct((n, 128), jnp.float32),
            jax.ShapeDtypeStruct((1, 128), jnp.float32),
            jax.ShapeDtypeStruct((1, 128), jnp.float32),
            jax.ShapeDtypeStruct((1, 128), jnp.float32),
        ],
    )(h, agg, w1, b1, eps)


def _pass_b(y1, m1, sd1, g1, bt1, w2, b2, n):
    nb = n // BN

    def kern(y1_ref, m_ref, sd_ref, g_ref, bt_ref, w2_ref, b2_ref, y2_ref,
             sh_ref, s_ref, q_ref):
        t = jnp.maximum((y1_ref[...] - m_ref[...]) / sd_ref[...] * g_ref[...]
                        + bt_ref[...], 0.0)
        y2 = jnp.dot(t, w2_ref[...], preferred_element_type=jnp.float32) + b2_ref[...]
        y2_ref[...] = y2

        @pl.when(pl.program_id(0) == 0)
        def _():
            sh_ref[...] = jnp.sum(y2, axis=0, keepdims=True) / BN
            s_ref[...] = jnp.zeros_like(s_ref)
            q_ref[...] = jnp.zeros_like(q_ref)

        yc = y2 - sh_ref[...]
        s_ref[...] += jnp.sum(yc, axis=0, keepdims=True)
        q_ref[...] += jnp.sum(yc * yc, axis=0, keepdims=True)

    return pl.pallas_call(
        kern,
        grid=(nb,),
        in_specs=[
            pl.BlockSpec((BN, 128), lambda i: (i, 0)),
            pl.BlockSpec((1, 128), lambda i: (0, 0)),
            pl.BlockSpec((1, 128), lambda i: (0, 0)),
            pl.BlockSpec((1, 128), lambda i: (0, 0)),
            pl.BlockSpec((1, 128), lambda i: (0, 0)),
            pl.BlockSpec((128, 64), lambda i: (0, 0)),
            pl.BlockSpec((1, 64), lambda i: (0, 0)),
        ],
        out_specs=[
            pl.BlockSpec((BN, 64), lambda i: (i, 0)),
            pl.BlockSpec((1, 64), lambda i: (0, 0)),
            pl.BlockSpec((1, 64), lambda i: (0, 0)),
            pl.BlockSpec((1, 64), lambda i: (0, 0)),
        ],
        out_shape=[
            jax.ShapeDtypeStruct((n, 64), jnp.float32),
            jax.ShapeDtypeStruct((1, 64), jnp.float32),
            jax.ShapeDtypeStruct((1, 64), jnp.float32),
            jax.ShapeDtypeStruct((1, 64), jnp.float32),
        ],
    )(y1, m1, sd1, g1, bt1, w2, b2)


def _finish_expand(y2, m2, sd2, g2, bt2, bond_half, n):
    nb = n // BN

    def kern(y2_ref, m_ref, sd_ref, g_ref, bt_ref, bond_ref, h_ref, t_ref):
        hb = jnp.maximum((y2_ref[...] - m_ref[...]) / sd_ref[...] * g_ref[...]
                         + bt_ref[...], 0.0)
        h_ref[...] = hb
        bond = bond_ref[...]
        for cb in range(10):
            half = hb[:, 32 * (cb // 5):32 * (cb // 5) + 32]
            t_ref[cb] = jnp.maximum(half + bond[cb][None, :], 0.0)

    return pl.pallas_call(
        kern,
        grid=(nb,),
        in_specs=[
            pl.BlockSpec((BN, 64), lambda i: (i, 0)),
            pl.BlockSpec((1, 64), lambda i: (0, 0)),
            pl.BlockSpec((1, 64), lambda i: (0, 0)),
            pl.BlockSpec((1, 64), lambda i: (0, 0)),
            pl.BlockSpec((1, 64), lambda i: (0, 0)),
            pl.BlockSpec((10, 32), lambda i: (0, 0)),
        ],
        out_specs=[
            pl.BlockSpec((BN, 64), lambda i: (i, 0)),
            pl.BlockSpec((10, BN, 32), lambda i: (0, i, 0)),
        ],
        out_shape=[
            jax.ShapeDtypeStruct((n, 64), jnp.float32),
            jax.ShapeDtypeStruct((10, n, 32), jnp.float32),
        ],
    )(y2, m2, sd2, g2, bt2, bond_half)


def _finish_readout(y2, m2, sd2, g2, bt2, batch3, n, g):
    nb = n // BN

    def kern(y2_ref, m_ref, sd_ref, g_ref, bt_ref, b_ref, sums_ref,
             counts_ref):
        hb = ((y2_ref[...] - m_ref[...]) / sd_ref[...] * g_ref[...]
              + bt_ref[...])
        bb = b_ref[0, 0, :]
        onehot = (lax.broadcasted_iota(jnp.int32, (g, BN), 0)
                  == bb[None, :]).astype(jnp.float32)

        @pl.when(pl.program_id(0) == 0)
        def _():
            sums_ref[...] = jnp.zeros_like(sums_ref)
            counts_ref[...] = jnp.zeros_like(counts_ref)

        sums_ref[...] += lax.dot_general(onehot, hb, (((1,), (0,)), ((), ())),
                                         preferred_element_type=jnp.float32, precision=lax.Precision.HIGHEST)
        counts_ref[...] += jnp.sum(onehot, axis=1)[None, :]

    return pl.pallas_call(
        kern,
        grid=(nb,),
        in_specs=[
            pl.BlockSpec((BN, 64), lambda i: (i, 0)),
            pl.BlockSpec((1, 64), lambda i: (0, 0)),
            pl.BlockSpec((1, 64), lambda i: (0, 0)),
            pl.BlockSpec((1, 64), lambda i: (0, 0)),
            pl.BlockSpec((1, 64), lambda i: (0, 0)),
            pl.BlockSpec((1, 1, BN), lambda i: (i, 0, 0)),
        ],
        out_specs=[
            pl.BlockSpec((g, 64), lambda i: (0, 0)),
            pl.BlockSpec((1, g), lambda i: (0, 0)),
        ],
        out_shape=[
            jax.ShapeDtypeStruct((g, 64), jnp.float32),
            jax.ShapeDtypeStruct((1, g), jnp.float32),
        ],
    )(y2, m2, sd2, g2, bt2, batch3)


def _bn_stats(sh, s, q, n):
    mw = s / n
    m = mw + sh
    v = q / n - mw * mw
    return m, jnp.sqrt(v + 1e-5)


def kernel(atom_emb, bond_tabs, W1s, b1s, g1s, bt1s, W2s, b2s, epss, g2s, bt2s,
           x, edge_attr, edge_index, batch):
    n = x.shape[0]
    e = edge_attr.shape[0]
    g = 512
    nlayers = bond_tabs.shape[0]

    x = x.astype(jnp.int32)
    src = edge_index[0].astype(jnp.int32)
    dst = edge_index[1].astype(jnp.int32)
    attr = edge_attr.astype(jnp.int32)
    batch = batch.astype(jnp.int32)

    gb = 8
    ng = -(-e // (NS * 128 * gb)) * gb
    nblk = ng // gb
    epad = NS * ng * 128 - e
    nacc = n + 176
    base = attr * n + dst
    pad_g = jnp.arange(epad, dtype=jnp.int32) % 1024
    pad_s = n + jnp.arange(epad, dtype=jnp.int32) % (nacc - n)
    base_p = jnp.concatenate([base, pad_g])
    src_p = jnp.concatenate([src, pad_s])
    idx2p = jnp.stack([base_p, base_p + 5 * n]).reshape(NC, NS, ng, 128)
    srcp = src_p.reshape(NS, ng, 128)

    x3 = x.reshape(n // BN, 1, BN)
    batch3 = batch.reshape(n // BN, 1, BN)

    h = None
    for l in range(nlayers):
        bond_half = jnp.concatenate(
            [bond_tabs[l][:, :32], bond_tabs[l][:, 32:]], axis=0)
        if l == 0:
            h, T = _expand0(x3, atom_emb, bond_half, n)
        else:
            h, T = _finish_expand(y2, m2, sd2, g2s[l - 1].reshape(1, -1),
                                  bt2s[l - 1].reshape(1, -1), bond_half, n)
        agg = _edge_pass(T.reshape(10 * n, 32), idx2p, srcp,
                         n, nacc, ng, gb, nblk)
        eps = epss[l].reshape(1, 1)
        y1, sh1, s1, q1 = _pass_a(h, agg, W1s[l], b1s[l].reshape(1, -1), eps, n)
        m1, sd1 = _bn_stats(sh1, s1, q1, n)
        y2, sh2, s2, q2 = _pass_b(y1, m1, sd1, g1s[l].reshape(1, -1),
                                  bt1s[l].reshape(1, -1),
                                  W2s[l], b2s[l].reshape(1, -1), n)
        m2, sd2 = _bn_stats(sh2, s2, q2, n)

    sums, counts = _finish_readout(y2, m2, sd2, g2s[3].reshape(1, -1),
                                   bt2s[3].reshape(1, -1), batch3, n, g)
    return sums / jnp.maximum(counts.reshape(g, 1), 1.0)


# speedup vs baseline: 8.1261x; 1.3584x over previous
import functools

import jax
import jax.numpy as jnp
from jax import lax
from jax.experimental import pallas as pl
from jax.experimental.pallas import tpu as pltpu
from jax.experimental.pallas import tpu_sc as plsc

NC = 2
NS = 16
BN = 2000


def _dot_x3(a, b):
    f = jnp.float32
    ah = a.astype(jnp.bfloat16)
    al = (a - ah.astype(f)).astype(jnp.bfloat16)
    bh = b.astype(jnp.bfloat16)
    bl = (b - bh.astype(f)).astype(jnp.bfloat16)
    dn = (((1,), (0,)), ((), ()))
    return (lax.dot_general(al, bh, dn, preferred_element_type=f)
            + lax.dot_general(ah, bl, dn, preferred_element_type=f)
            + lax.dot_general(ah, bh, dn, preferred_element_type=f))


def _edge_pass(T_flat, idx2p, srcp, n, nacc, ng, gb, nblk):
    mesh = plsc.VectorSubcoreMesh(
        core_axis_name="c", subcore_axis_name="s", num_cores=NC, num_subcores=NS
    )
    nz = nacc // NS
    ntail = n - (NS - 1) * nz
    assert 0 < ntail <= nz and ntail % 8 == 0 and nz % 8 == 0

    nslot = 4

    @functools.partial(
        pl.kernel,
        out_type=jax.ShapeDtypeStruct((NC, n, 32), jnp.float32),
        mesh=mesh,
        compiler_params=pltpu.CompilerParams(use_tc_tiling_on_sc=False),
        scratch_types=[
            pltpu.VMEM((gb, 128), jnp.int32),
            pltpu.VMEM((gb, 128), jnp.int32),
            pltpu.VMEM((nslot * 128, 32), jnp.float32),
            pltpu.VMEM_SHARED((nacc, 32), jnp.float32),
            pltpu.SemaphoreType.DMA,
            pltpu.SemaphoreType.DMA,
        ],
    )
    def edge_kernel(t_hbm, idx_hbm, src_hbm, agg_hbm,
                    idx_blk, src_blk, rows, acc, gsem, ssem):
        c = lax.axis_index("c")
        s = lax.axis_index("s")

        zero16 = jnp.zeros((16,), jnp.float32)

        def zrow(i, carry):
            rows[i, pl.ds(0, 16)] = zero16
            rows[i, pl.ds(16, 16)] = zero16
            return carry

        lax.fori_loop(0, nslot * 128, zrow, 0)

        nzc = nslot * 128

        def zacc(k, carry):
            pltpu.sync_copy(rows, acc.at[pl.ds(s * nz + k * nzc, nzc)])
            return carry

        lax.fori_loop(0, nz // nzc, zacc, 0)
        if nz % nzc:
            pltpu.sync_copy(rows.at[pl.ds(0, nz % nzc)],
                            acc.at[pl.ds(s * nz + (nz // nzc) * nzc, nz % nzc)])
        plsc.subcore_barrier()

        def block(bi, carry):
            g0 = bi * gb
            pltpu.sync_copy(idx_hbm.at[c, s, pl.ds(g0, gb)], idx_blk)
            pltpu.sync_copy(src_hbm.at[s, pl.ds(g0, gb)], src_blk)
            for half in range(gb // nslot):
                gathers = [
                    pltpu.async_copy(t_hbm.at[idx_blk.at[half * nslot + j]],
                                     rows.at[pl.ds(j * 128, 128)], gsem)
                    for j in range(nslot)
                ]
                for d in gathers:
                    d.wait()
                scatters = [
                    pltpu.async_copy(rows.at[pl.ds(j * 128, 128)],
                                     acc.at[src_blk.at[half * nslot + j]],
                                     ssem, add=True)
                    for j in range(nslot)
                ]
                for d in scatters:
                    d.wait()
            return carry

        lax.fori_loop(0, nblk, block, 0)
        plsc.subcore_barrier()
        pltpu.sync_copy(acc.at[pl.ds(s * nz, ntail)],
                        agg_hbm.at[c, pl.ds(s * nz, ntail)])

        @pl.when(s < NS - 1)
        def _():
            pltpu.sync_copy(acc.at[pl.ds(s * nz + ntail, nz - ntail)],
                            agg_hbm.at[c, pl.ds(s * nz + ntail, nz - ntail)])

    return edge_kernel(T_flat, idx2p, srcp)


def _expand0(x3, atom_emb, bond_half, n):
    nb = x3.shape[0]
    av, d = atom_emb.shape

    def kern(x_ref, emb_ref, bond_ref, h_ref, t_ref):
        xb = x_ref[0, 0, :]
        onehot = (lax.broadcasted_iota(jnp.int32, (av, BN), 0)
                  == xb[None, :]).astype(jnp.float32)
        hb = lax.dot_general(onehot, emb_ref[...], (((0,), (0,)), ((), ())),
                             preferred_element_type=jnp.float32,
                             precision=lax.Precision.HIGHEST)
        h_ref[...] = hb
        bond = bond_ref[...]
        parts = []
        for cb in range(10):
            half = hb[:, 32 * (cb // 5):32 * (cb // 5) + 32]
            tb = jnp.maximum(half + bond[cb][None, :], 0.0)
            q = BN // 4
            parts.append(jnp.concatenate(
                [tb[k * q:(k + 1) * q, :] for k in range(4)], axis=1))
        t_ref[...] = jnp.concatenate(parts, axis=0)

    return pl.pallas_call(
        kern,
        grid=(nb,),
        in_specs=[
            pl.BlockSpec((1, 1, BN), lambda i: (i, 0, 0)),
            pl.BlockSpec((av, d), lambda i: (0, 0)),
            pl.BlockSpec((10, 32), lambda i: (0, 0)),
        ],
        out_specs=[
            pl.BlockSpec((BN, d), lambda i: (i, 0)),
            pl.BlockSpec((10 * BN // 4, 128), lambda i: (i, 0)),
        ],
        out_shape=[
            jax.ShapeDtypeStruct((n, d), jnp.float32),
            jax.ShapeDtypeStruct((10 * n // 4, 128), jnp.float32),
        ],
    )(x3, atom_emb, bond_half)


def _pass_a(h, agg, w1, b1, eps, n):
    nb = n // BN

    def kern(h_ref, agg_ref, w1_ref, b1_ref, eps_ref, y1_ref, sh_ref, s_ref,
             q_ref):
        e = 1.0 + eps_ref[0, 0]
        z = e * h_ref[...] + jnp.concatenate([agg_ref[0], agg_ref[1]], axis=1)
        y1 = jnp.dot(z, w1_ref[...], preferred_element_type=jnp.float32) + b1_ref[...]
        y1_ref[...] = y1

        @pl.when(pl.program_id(0) == 0)
        def _():
            sh_ref[...] = jnp.sum(y1, axis=0, keepdims=True) / BN
            s_ref[...] = jnp.zeros_like(s_ref)
            q_ref[...] = jnp.zeros_like(q_ref)

        yc = y1 - sh_ref[...]
        s_ref[...] += jnp.sum(yc, axis=0, keepdims=True)
        q_ref[...] += jnp.sum(yc * yc, axis=0, keepdims=True)

    return pl.pallas_call(
        kern,
        grid=(nb,),
        in_specs=[
            pl.BlockSpec((BN, 64), lambda i: (i, 0)),
            pl.BlockSpec((2, BN, 32), lambda i: (0, i, 0)),
            pl.BlockSpec((64, 128), lambda i: (0, 0)),
            pl.BlockSpec((1, 128), lambda i: (0, 0)),
            pl.BlockSpec((1, 1), lambda i: (0, 0)),
        ],
        out_specs=[
            pl.BlockSpec((BN, 128), lambda i: (i, 0)),
            pl.BlockSpec((1, 128), lambda i: (0, 0)),
            pl.BlockSpec((1, 128), lambda i: (0, 0)),
            pl.BlockSpec((1, 128), lambda i: (0, 0)),
        ],
        out_shape=[
            jax.ShapeDtypeStruct((n, 128), jnp.float32),
            jax.ShapeDtypeStruct((1, 128), jnp.float32),
            jax.ShapeDtypeStruct((1, 128), jnp.float32),
            jax.ShapeDtypeStruct((1, 128), jnp.float32),
        ],
    )(h, agg, w1, b1, eps)


def _pass_b(y1, m1, sd1, g1, bt1, w2, b2, n):
    nb = n // BN

    def kern(y1_ref, m_ref, sd_ref, g_ref, bt_ref, w2_ref, b2_ref, y2_ref,
             sh_ref, s_ref, q_ref):
        t = jnp.maximum((y1_ref[...] - m_ref[...]) / sd_ref[...] * g_ref[...]
                        + bt_ref[...], 0.0)
        y2 = jnp.dot(t, w2_ref[...], preferred_element_type=jnp.float32) + b2_ref[...]
        y2_ref[...] = y2

        @pl.when(pl.program_id(0) == 0)
        def _():
            sh_ref[...] = jnp.sum(y2, axis=0, keepdims=True) / BN
            s_ref[...] = jnp.zeros_like(s_ref)
            q_ref[...] = jnp.zeros_like(q_ref)

        yc = y2 - sh_ref[...]
        s_ref[...] += jnp.sum(yc, axis=0, keepdims=True)
        q_ref[...] += jnp.sum(yc * yc, axis=0, keepdims=True)

    return pl.pallas_call(
        kern,
        grid=(nb,),
        in_specs=[
            pl.BlockSpec((BN, 128), lambda i: (i, 0)),
            pl.BlockSpec((1, 128), lambda i: (0, 0)),
            pl.BlockSpec((1, 128), lambda i: (0, 0)),
            pl.BlockSpec((1, 128), lambda i: (0, 0)),
            pl.BlockSpec((1, 128), lambda i: (0, 0)),
            pl.BlockSpec((128, 64), lambda i: (0, 0)),
            pl.BlockSpec((1, 64), lambda i: (0, 0)),
        ],
        out_specs=[
            pl.BlockSpec((BN, 64), lambda i: (i, 0)),
            pl.BlockSpec((1, 64), lambda i: (0, 0)),
            pl.BlockSpec((1, 64), lambda i: (0, 0)),
            pl.BlockSpec((1, 64), lambda i: (0, 0)),
        ],
        out_shape=[
            jax.ShapeDtypeStruct((n, 64), jnp.float32),
            jax.ShapeDtypeStruct((1, 64), jnp.float32),
            jax.ShapeDtypeStruct((1, 64), jnp.float32),
            jax.ShapeDtypeStruct((1, 64), jnp.float32),
        ],
    )(y1, m1, sd1, g1, bt1, w2, b2)


def _finish_expand(y2, m2, sd2, g2, bt2, bond_half, n):
    nb = n // BN

    def kern(y2_ref, m_ref, sd_ref, g_ref, bt_ref, bond_ref, h_ref, t_ref):
        hb = jnp.maximum((y2_ref[...] - m_ref[...]) / sd_ref[...]
                         * g_ref[...] + bt_ref[...], 0.0)
        h_ref[...] = hb
        bond = bond_ref[...]
        parts = []
        for cb in range(10):
            half = hb[:, 32 * (cb // 5):32 * (cb // 5) + 32]
            tb = jnp.maximum(half + bond[cb][None, :], 0.0)
            q = BN // 4
            parts.append(jnp.concatenate(
                [tb[k * q:(k + 1) * q, :] for k in range(4)], axis=1))
        t_ref[...] = jnp.concatenate(parts, axis=0)

    return pl.pallas_call(
        kern,
        grid=(nb,),
        in_specs=[
            pl.BlockSpec((BN, 64), lambda i: (i, 0)),
            pl.BlockSpec((1, 64), lambda i: (0, 0)),
            pl.BlockSpec((1, 64), lambda i: (0, 0)),
            pl.BlockSpec((1, 64), lambda i: (0, 0)),
            pl.BlockSpec((1, 64), lambda i: (0, 0)),
            pl.BlockSpec((10, 32), lambda i: (0, 0)),
        ],
        out_specs=[
            pl.BlockSpec((BN, 64), lambda i: (i, 0)),
            pl.BlockSpec((10 * BN // 4, 128), lambda i: (i, 0)),
        ],
        out_shape=[
            jax.ShapeDtypeStruct((n, 64), jnp.float32),
            jax.ShapeDtypeStruct((10 * n // 4, 128), jnp.float32),
        ],
    )(y2, m2, sd2, g2, bt2, bond_half)


def _finish_readout(y2, m2, sd2, g2, bt2, batch3, n, g):
    nb = n // BN

    def kern(y2_ref, m_ref, sd_ref, g_ref, bt_ref, b_ref, sums_ref,
             counts_ref):
        hb = ((y2_ref[...] - m_ref[...]) / sd_ref[...] * g_ref[...]
              + bt_ref[...])
        bb = b_ref[0, 0, :]
        onehot = (lax.broadcasted_iota(jnp.int32, (g, BN), 0)
                  == bb[None, :]).astype(jnp.float32)

        @pl.when(pl.program_id(0) == 0)
        def _():
            sums_ref[...] = jnp.zeros_like(sums_ref)
            counts_ref[...] = jnp.zeros_like(counts_ref)

        sums_ref[...] += lax.dot_general(onehot, hb, (((1,), (0,)), ((), ())),
                                         preferred_element_type=jnp.float32, precision=lax.Precision.HIGHEST)
        counts_ref[...] += jnp.sum(onehot, axis=1)[None, :]

    return pl.pallas_call(
        kern,
        grid=(nb,),
        in_specs=[
            pl.BlockSpec((BN, 64), lambda i: (i, 0)),
            pl.BlockSpec((1, 64), lambda i: (0, 0)),
            pl.BlockSpec((1, 64), lambda i: (0, 0)),
            pl.BlockSpec((1, 64), lambda i: (0, 0)),
            pl.BlockSpec((1, 64), lambda i: (0, 0)),
            pl.BlockSpec((1, 1, BN), lambda i: (i, 0, 0)),
        ],
        out_specs=[
            pl.BlockSpec((g, 64), lambda i: (0, 0)),
            pl.BlockSpec((1, g), lambda i: (0, 0)),
        ],
        out_shape=[
            jax.ShapeDtypeStruct((g, 64), jnp.float32),
            jax.ShapeDtypeStruct((1, g), jnp.float32),
        ],
    )(y2, m2, sd2, g2, bt2, batch3)


def _bn_stats(sh, s, q, n):
    mw = s / n
    m = mw + sh
    v = q / n - mw * mw
    return m, jnp.sqrt(v + 1e-5)


def kernel(atom_emb, bond_tabs, W1s, b1s, g1s, bt1s, W2s, b2s, epss, g2s, bt2s,
           x, edge_attr, edge_index, batch):
    n = x.shape[0]
    e = edge_attr.shape[0]
    g = 512
    nlayers = bond_tabs.shape[0]

    x = x.astype(jnp.int32)
    src = edge_index[0].astype(jnp.int32)
    dst = edge_index[1].astype(jnp.int32)
    attr = edge_attr.astype(jnp.int32)
    batch = batch.astype(jnp.int32)

    gb = 8
    ng = -(-e // (NS * 128 * gb)) * gb
    nblk = ng // gb
    epad = NS * ng * 128 - e
    nacc = n + 176
    o = dst % BN
    base = (dst // BN) * (10 * BN) + attr * BN + (o % (BN // 4)) * 4 + o // (BN // 4)
    pad_g = jnp.arange(epad, dtype=jnp.int32) % 1024
    pad_s = n + jnp.arange(epad, dtype=jnp.int32) % (nacc - n)
    base_p = jnp.concatenate([base, pad_g])
    src_p = jnp.concatenate([src, pad_s])
    idx2p = jnp.stack([base_p, base_p + 5 * BN]).reshape(NC, NS, ng, 128)
    srcp = src_p.reshape(NS, ng, 128)

    x3 = x.reshape(n // BN, 1, BN)
    batch3 = batch.reshape(n // BN, 1, BN)

    h = None
    for l in range(nlayers):
        bond_half = jnp.concatenate(
            [bond_tabs[l][:, :32], bond_tabs[l][:, 32:]], axis=0)
        if l == 0:
            h, T = _expand0(x3, atom_emb, bond_half, n)
        else:
            h, T = _finish_expand(y2, m2, sd2, g2s[l - 1].reshape(1, -1),
                                  bt2s[l - 1].reshape(1, -1), bond_half, n)
        agg = _edge_pass(T.reshape(10 * n, 32), idx2p, srcp,
                         n, nacc, ng, gb, nblk)
        eps = epss[l].reshape(1, 1)
        y1, sh1, s1, q1 = _pass_a(h, agg, W1s[l], b1s[l].reshape(1, -1), eps, n)
        m1, sd1 = _bn_stats(sh1, s1, q1, n)
        y2, sh2, s2, q2 = _pass_b(y1, m1, sd1, g1s[l].reshape(1, -1),
                                  bt1s[l].reshape(1, -1),
                                  W2s[l], b2s[l].reshape(1, -1), n)
        m2, sd2 = _bn_stats(sh2, s2, q2, n)

    sums, counts = _finish_readout(y2, m2, sd2, g2s[3].reshape(1, -1),
                                   bt2s[3].reshape(1, -1), batch3, n, g)
    return sums / jnp.maximum(counts.reshape(g, 1), 1.0)


# speedup vs baseline: 8.7469x; 1.0764x over previous
import functools

import jax
import jax.numpy as jnp
from jax import lax
from jax.experimental import pallas as pl
from jax.experimental.pallas import tpu as pltpu
from jax.experimental.pallas import tpu_sc as plsc

NC = 2
NS = 16
BN = 2000


def _edge_pass(T_flat, idx2p, srcp, n, nacc, ng, gb, nblk):
    mesh = plsc.VectorSubcoreMesh(
        core_axis_name="c", subcore_axis_name="s", num_cores=NC, num_subcores=NS
    )
    nz = nacc // NS
    ntail = n - (NS - 1) * nz
    assert 0 < ntail <= nz and ntail % 8 == 0 and nz % 8 == 0

    nslot = 6

    @functools.partial(
        pl.kernel,
        out_type=jax.ShapeDtypeStruct((NC, n, 32), jnp.float32),
        mesh=mesh,
        compiler_params=pltpu.CompilerParams(use_tc_tiling_on_sc=False),
        scratch_types=[
            pltpu.VMEM((gb, 128), jnp.int32),
            pltpu.VMEM((gb, 128), jnp.int32),
            pltpu.VMEM((nslot * 128, 32), jnp.float32),
            pltpu.VMEM_SHARED((nacc, 32), jnp.float32),
        ] + [pltpu.SemaphoreType.DMA] * (2 * nslot),
    )
    def edge_kernel(t_hbm, idx_hbm, src_hbm, agg_hbm,
                    idx_blk, src_blk, rows, acc, *sems):
        gsems = sems[:nslot]
        ssems = sems[nslot:]
        c = lax.axis_index("c")
        s = lax.axis_index("s")

        zero16 = jnp.zeros((16,), jnp.float32)

        def zrow(i, carry):
            rows[i, pl.ds(0, 16)] = zero16
            rows[i, pl.ds(16, 16)] = zero16
            return carry

        lax.fori_loop(0, nslot * 128, zrow, 0)

        nzc = nslot * 128

        def zacc(k, carry):
            pltpu.sync_copy(rows, acc.at[pl.ds(s * nz + k * nzc, nzc)])
            return carry

        lax.fori_loop(0, nz // nzc, zacc, 0)
        if nz % nzc:
            pltpu.sync_copy(rows.at[pl.ds(0, nz % nzc)],
                            acc.at[pl.ds(s * nz + (nz // nzc) * nzc, nz % nzc)])
        plsc.subcore_barrier()

        def slot_rows(k):
            return rows.at[pl.ds(k * 128, 128)]

        def drain_scatter(k):
            pltpu.make_async_copy(slot_rows(k), acc.at[src_blk.at[0]],
                                  ssems[k]).wait()

        def block_body(bi, drain_first):
            if drain_first:
                for k in range(nslot):
                    drain_scatter(k)
            g0 = bi * gb
            pltpu.sync_copy(idx_hbm.at[c, s, pl.ds(g0, gb)], idx_blk)
            pltpu.sync_copy(src_hbm.at[s, pl.ds(g0, gb)], src_blk)
            g = {}
            for j in range(nslot):
                g[j] = pltpu.async_copy(t_hbm.at[idx_blk.at[j]],
                                        slot_rows(j), gsems[j])
            for j in range(gb):
                k = j % nslot
                g[j].wait()
                sd = pltpu.async_copy(slot_rows(k), acc.at[src_blk.at[j]],
                                      ssems[k], add=True)
                if j + nslot < gb:
                    sd.wait()
                    g[j + nslot] = pltpu.async_copy(
                        t_hbm.at[idx_blk.at[j + nslot]], slot_rows(k),
                        gsems[k])

        block_body(0, False)

        def block(bi, carry):
            block_body(bi, True)
            return carry

        lax.fori_loop(1, nblk, block, 0)
        for k in range(nslot):
            drain_scatter(k)
        plsc.subcore_barrier()
        pltpu.sync_copy(acc.at[pl.ds(s * nz, ntail)],
                        agg_hbm.at[c, pl.ds(s * nz, ntail)])

        @pl.when(s < NS - 1)
        def _():
            pltpu.sync_copy(acc.at[pl.ds(s * nz + ntail, nz - ntail)],
                            agg_hbm.at[c, pl.ds(s * nz + ntail, nz - ntail)])

    return edge_kernel(T_flat, idx2p, srcp)


def _expand0(x3, atom_emb, bond_half, n):
    nb = x3.shape[0]
    av, d = atom_emb.shape

    def kern(x_ref, emb_ref, bond_ref, h_ref, t_ref):
        xb = x_ref[0, 0, :]
        onehot = (lax.broadcasted_iota(jnp.int32, (av, BN), 0)
                  == xb[None, :]).astype(jnp.float32)
        hb = lax.dot_general(onehot, emb_ref[...], (((0,), (0,)), ((), ())),
                             preferred_element_type=jnp.float32,
                             precision=lax.Precision.HIGHEST)
        h_ref[...] = hb
        bond = bond_ref[...]
        parts = []
        for cb in range(10):
            half = hb[:, 32 * (cb // 5):32 * (cb // 5) + 32]
            tb = jnp.maximum(half + bond[cb][None, :], 0.0)
            q = BN // 4
            parts.append(jnp.concatenate(
                [tb[k * q:(k + 1) * q, :] for k in range(4)], axis=1))
        t_ref[...] = jnp.concatenate(parts, axis=0)

    return pl.pallas_call(
        kern,
        grid=(nb,),
        in_specs=[
            pl.BlockSpec((1, 1, BN), lambda i: (i, 0, 0)),
            pl.BlockSpec((av, d), lambda i: (0, 0)),
            pl.BlockSpec((10, 32), lambda i: (0, 0)),
        ],
        out_specs=[
            pl.BlockSpec((BN, d), lambda i: (i, 0)),
            pl.BlockSpec((10 * BN // 4, 128), lambda i: (i, 0)),
        ],
        out_shape=[
            jax.ShapeDtypeStruct((n, d), jnp.float32),
            jax.ShapeDtypeStruct((10 * n // 4, 128), jnp.float32),
        ],
    )(x3, atom_emb, bond_half)


def _pass_a(h, agg, w1, b1, eps, n):
    nb = n // BN

    def kern(h_ref, agg_ref, w1_ref, b1_ref, eps_ref, y1_ref, sh_ref, s_ref,
             q_ref):
        e = 1.0 + eps_ref[0, 0]
        z = e * h_ref[...] + jnp.concatenate([agg_ref[0], agg_ref[1]], axis=1)
        y1 = jnp.dot(z, w1_ref[...], preferred_element_type=jnp.float32) + b1_ref[...]
        y1_ref[...] = y1

        @pl.when(pl.program_id(0) == 0)
        def _():
            sh_ref[...] = jnp.sum(y1, axis=0, keepdims=True) / BN
            s_ref[...] = jnp.zeros_like(s_ref)
            q_ref[...] = jnp.zeros_like(q_ref)

        yc = y1 - sh_ref[...]
        s_ref[...] += jnp.sum(yc, axis=0, keepdims=True)
        q_ref[...] += jnp.sum(yc * yc, axis=0, keepdims=True)

    return pl.pallas_call(
        kern,
        grid=(nb,),
        in_specs=[
            pl.BlockSpec((BN, 64), lambda i: (i, 0)),
            pl.BlockSpec((2, BN, 32), lambda i: (0, i, 0)),
            pl.BlockSpec((64, 128), lambda i: (0, 0)),
            pl.BlockSpec((1, 128), lambda i: (0, 0)),
            pl.BlockSpec((1, 1), lambda i: (0, 0)),
        ],
        out_specs=[
            pl.BlockSpec((BN, 128), lambda i: (i, 0)),
            pl.BlockSpec((1, 128), lambda i: (0, 0)),
            pl.BlockSpec((1, 128), lambda i: (0, 0)),
            pl.BlockSpec((1, 128), lambda i: (0, 0)),
        ],
        out_shape=[
            jax.ShapeDtypeStruct((n, 128), jnp.float32),
            jax.ShapeDtypeStruct((1, 128), jnp.float32),
            jax.ShapeDtypeStruct((1, 128), jnp.float32),
            jax.ShapeDtypeStruct((1, 128), jnp.float32),
        ],
    )(h, agg, w1, b1, eps)


def _pass_b(y1, m1, sd1, g1, bt1, w2, b2, n):
    nb = n // BN

    def kern(y1_ref, m_ref, sd_ref, g_ref, bt_ref, w2_ref, b2_ref, y2_ref,
             sh_ref, s_ref, q_ref):
        t = jnp.maximum((y1_ref[...] - m_ref[...]) / sd_ref[...] * g_ref[...]
                        + bt_ref[...], 0.0)
        y2 = jnp.dot(t, w2_ref[...], preferred_element_type=jnp.float32) + b2_ref[...]
        y2_ref[...] = y2

        @pl.when(pl.program_id(0) == 0)
        def _():
            sh_ref[...] = jnp.sum(y2, axis=0, keepdims=True) / BN
            s_ref[...] = jnp.zeros_like(s_ref)
            q_ref[...] = jnp.zeros_like(q_ref)

        yc = y2 - sh_ref[...]
        s_ref[...] += jnp.sum(yc, axis=0, keepdims=True)
        q_ref[...] += jnp.sum(yc * yc, axis=0, keepdims=True)

    return pl.pallas_call(
        kern,
        grid=(nb,),
        in_specs=[
            pl.BlockSpec((BN, 128), lambda i: (i, 0)),
            pl.BlockSpec((1, 128), lambda i: (0, 0)),
            pl.BlockSpec((1, 128), lambda i: (0, 0)),
            pl.BlockSpec((1, 128), lambda i: (0, 0)),
            pl.BlockSpec((1, 128), lambda i: (0, 0)),
            pl.BlockSpec((128, 64), lambda i: (0, 0)),
            pl.BlockSpec((1, 64), lambda i: (0, 0)),
        ],
        out_specs=[
            pl.BlockSpec((BN, 64), lambda i: (i, 0)),
            pl.BlockSpec((1, 64), lambda i: (0, 0)),
            pl.BlockSpec((1, 64), lambda i: (0, 0)),
            pl.BlockSpec((1, 64), lambda i: (0, 0)),
        ],
        out_shape=[
            jax.ShapeDtypeStruct((n, 64), jnp.float32),
            jax.ShapeDtypeStruct((1, 64), jnp.float32),
            jax.ShapeDtypeStruct((1, 64), jnp.float32),
            jax.ShapeDtypeStruct((1, 64), jnp.float32),
        ],
    )(y1, m1, sd1, g1, bt1, w2, b2)


def _finish_expand(y2, m2, sd2, g2, bt2, bond_half, n):
    nb = n // BN

    def kern(y2_ref, m_ref, sd_ref, g_ref, bt_ref, bond_ref, h_ref, t_ref):
        hb = jnp.maximum((y2_ref[...] - m_ref[...]) / sd_ref[...]
                         * g_ref[...] + bt_ref[...], 0.0)
        h_ref[...] = hb
        bond = bond_ref[...]
        parts = []
        for cb in range(10):
            half = hb[:, 32 * (cb // 5):32 * (cb // 5) + 32]
            tb = jnp.maximum(half + bond[cb][None, :], 0.0)
            q = BN // 4
            parts.append(jnp.concatenate(
                [tb[k * q:(k + 1) * q, :] for k in range(4)], axis=1))
        t_ref[...] = jnp.concatenate(parts, axis=0)

    return pl.pallas_call(
        kern,
        grid=(nb,),
        in_specs=[
            pl.BlockSpec((BN, 64), lambda i: (i, 0)),
            pl.BlockSpec((1, 64), lambda i: (0, 0)),
            pl.BlockSpec((1, 64), lambda i: (0, 0)),
            pl.BlockSpec((1, 64), lambda i: (0, 0)),
            pl.BlockSpec((1, 64), lambda i: (0, 0)),
            pl.BlockSpec((10, 32), lambda i: (0, 0)),
        ],
        out_specs=[
            pl.BlockSpec((BN, 64), lambda i: (i, 0)),
            pl.BlockSpec((10 * BN // 4, 128), lambda i: (i, 0)),
        ],
        out_shape=[
            jax.ShapeDtypeStruct((n, 64), jnp.float32),
            jax.ShapeDtypeStruct((10 * n // 4, 128), jnp.float32),
        ],
    )(y2, m2, sd2, g2, bt2, bond_half)


def _finish_readout(y2, m2, sd2, g2, bt2, batch3, n, g):
    nb = n // BN

    def kern(y2_ref, m_ref, sd_ref, g_ref, bt_ref, b_ref, sums_ref,
             counts_ref):
        hb = ((y2_ref[...] - m_ref[...]) / sd_ref[...] * g_ref[...]
              + bt_ref[...])
        bb = b_ref[0, 0, :]
        onehot = (lax.broadcasted_iota(jnp.int32, (g, BN), 0)
                  == bb[None, :]).astype(jnp.float32)

        @pl.when(pl.program_id(0) == 0)
        def _():
            sums_ref[...] = jnp.zeros_like(sums_ref)
            counts_ref[...] = jnp.zeros_like(counts_ref)

        sums_ref[...] += lax.dot_general(onehot, hb, (((1,), (0,)), ((), ())),
                                         preferred_element_type=jnp.float32, precision=lax.Precision.HIGHEST)
        counts_ref[...] += jnp.sum(onehot, axis=1)[None, :]

    return pl.pallas_call(
        kern,
        grid=(nb,),
        in_specs=[
            pl.BlockSpec((BN, 64), lambda i: (i, 0)),
            pl.BlockSpec((1, 64), lambda i: (0, 0)),
            pl.BlockSpec((1, 64), lambda i: (0, 0)),
            pl.BlockSpec((1, 64), lambda i: (0, 0)),
            pl.BlockSpec((1, 64), lambda i: (0, 0)),
            pl.BlockSpec((1, 1, BN), lambda i: (i, 0, 0)),
        ],
        out_specs=[
            pl.BlockSpec((g, 64), lambda i: (0, 0)),
            pl.BlockSpec((1, g), lambda i: (0, 0)),
        ],
        out_shape=[
            jax.ShapeDtypeStruct((g, 64), jnp.float32),
            jax.ShapeDtypeStruct((1, g), jnp.float32),
        ],
    )(y2, m2, sd2, g2, bt2, batch3)


def _bn_stats(sh, s, q, n):
    mw = s / n
    m = mw + sh
    v = q / n - mw * mw
    return m, jnp.sqrt(v + 1e-5)


def kernel(atom_emb, bond_tabs, W1s, b1s, g1s, bt1s, W2s, b2s, epss, g2s, bt2s,
           x, edge_attr, edge_index, batch):
    n = x.shape[0]
    e = edge_attr.shape[0]
    g = 512
    nlayers = bond_tabs.shape[0]

    x = x.astype(jnp.int32)
    src = edge_index[0].astype(jnp.int32)
    dst = edge_index[1].astype(jnp.int32)
    attr = edge_attr.astype(jnp.int32)
    batch = batch.astype(jnp.int32)

    gb = 8
    ng = -(-e // (NS * 128 * gb)) * gb
    nblk = ng // gb
    epad = NS * ng * 128 - e
    nacc = n + 176
    o = dst % BN
    base = (dst // BN) * (10 * BN) + attr * BN + (o % (BN // 4)) * 4 + o // (BN // 4)
    pad_g = jnp.arange(epad, dtype=jnp.int32) % 1024
    pad_s = n + jnp.arange(epad, dtype=jnp.int32) % (nacc - n)
    base_p = jnp.concatenate([base, pad_g])
    src_p = jnp.concatenate([src, pad_s])
    idx2p = jnp.stack([base_p, base_p + 5 * BN]).reshape(NC, NS, ng, 128)
    srcp = src_p.reshape(NS, ng, 128)

    x3 = x.reshape(n // BN, 1, BN)
    batch3 = batch.reshape(n // BN, 1, BN)

    h = None
    for l in range(nlayers):
        bond_half = jnp.concatenate(
            [bond_tabs[l][:, :32], bond_tabs[l][:, 32:]], axis=0)
        if l == 0:
            h, T = _expand0(x3, atom_emb, bond_half, n)
        else:
            h, T = _finish_expand(y2, m2, sd2, g2s[l - 1].reshape(1, -1),
                                  bt2s[l - 1].reshape(1, -1), bond_half, n)
        agg = _edge_pass(T.reshape(10 * n, 32), idx2p, srcp,
                         n, nacc, ng, gb, nblk)
        eps = epss[l].reshape(1, 1)
        y1, sh1, s1, q1 = _pass_a(h, agg, W1s[l], b1s[l].reshape(1, -1), eps, n)
        m1, sd1 = _bn_stats(sh1, s1, q1, n)
        y2, sh2, s2, q2 = _pass_b(y1, m1, sd1, g1s[l].reshape(1, -1),
                                  bt1s[l].reshape(1, -1),
                                  W2s[l], b2s[l].reshape(1, -1), n)
        m2, sd2 = _bn_stats(sh2, s2, q2, n)

    sums, counts = _finish_readout(y2, m2, sd2, g2s[3].reshape(1, -1),
                                   bt2s[3].reshape(1, -1), batch3, n, g)
    return sums / jnp.maximum(counts.reshape(g, 1), 1.0)
